# Initial kernel scaffold; baseline (speedup 1.0000x reference)
import jax
import jax.numpy as jnp
from jax.experimental import pallas as pl


def kernel(x, edge_index, edge_weight, W1, b1, W2, b2, We, be):
    raise NotImplementedError("write your pallas kernel here")


# speedup vs baseline: 6.9409x; 6.9409x over previous
import dataclasses
import functools

import jax
import jax.numpy as jnp
from jax import lax
from jax.experimental import pallas as pl
from jax.experimental.pallas import tpu as pltpu
from jax.experimental.pallas import tpu_sc as plsc

N = 50000
NP = 50176
NPT = NP // 16
E = 800000
EP = 802816
NCK = EP // 1024
NCKT = NCK // 16
RB = 1024

_mesh = plsc.VectorSubcoreMesh(core_axis_name="c", subcore_axis_name="s")

_sc_params = pltpu.CompilerParams()
if "needs_layout_passes" in pltpu.CompilerParams.__dataclass_fields__:
  _sc_params = dataclasses.replace(_sc_params, needs_layout_passes=False)


def _sc_deg(col3d, ew3d):
  @functools.partial(
      pl.kernel,
      out_type=jax.ShapeDtypeStruct((2 * NP,), jnp.float32),
      mesh=_mesh,
      scratch_types=[
          pltpu.VMEM((8, 128), jnp.int32),
          pltpu.VMEM((8, 128), jnp.float32),
          pltpu.VMEM((NPT,), jnp.float32),
          pltpu.VMEM_SHARED((NP,), jnp.float32),
      ],
      compiler_params=_sc_params,
  )
  def k(col_h, ew_h, out_h, colb, ewb, zb, acc):
    c = lax.axis_index("c")
    s = lax.axis_index("s")
    zv = jnp.zeros((16,), jnp.float32)

    @pl.loop(0, NPT // 16)
    def _(i):
      zb.at[pl.ds(i * 16, 16)][...] = zv

    pltpu.sync_copy(zb, acc.at[pl.ds(s * NPT, NPT)])
    plsc.subcore_barrier()

    @pl.loop(0, NCKT)
    def _(i):
      b = s * NCKT + i
      pltpu.sync_copy(col_h.at[b], colb)
      pltpu.sync_copy(ew_h.at[b], ewb)
      for j in range(8):
        pltpu.sync_copy(ewb.at[j], acc.at[colb.at[j]], add=True)

    plsc.subcore_barrier()
    pltpu.sync_copy(acc.at[pl.ds(s * NPT, NPT)], zb)
    pltpu.sync_copy(zb, out_h.at[pl.ds(c * NP + s * NPT, NPT)])

  return k(col3d, ew3d)


def _sc_agg(FPC, G, tabT, row3d, col3d, ew3d):
  NG = FPC // G

  @functools.partial(
      pl.kernel,
      out_type=jax.ShapeDtypeStruct((2 * FPC * NP,), jnp.float32),
      mesh=_mesh,
      scratch_types=[
          pltpu.VMEM((8, 128), jnp.int32),
          pltpu.VMEM((8, 128), jnp.int32),
          pltpu.VMEM((8, 128), jnp.float32),
          pltpu.VMEM((8, 128), jnp.float32),
          pltpu.VMEM((NPT,), jnp.float32),
          pltpu.VMEM_SHARED((G * NP,), jnp.float32),
          pltpu.VMEM_SHARED((G * NP,), jnp.float32),
          pltpu.SemaphoreType.DMA,
      ],
      compiler_params=_sc_params,
  )
  def k(tab_h, row_h, col_h, ew_h, out_h, srcb, dstb, ewb, valb, zb, tabs,
        accs, sem):
    c = lax.axis_index("c")
    s = lax.axis_index("s")

    @pl.loop(0, NG)
    def _(ng):
      fbase = c * FPC + ng * G
      for g in range(G):
        pltpu.sync_copy(tab_h.at[pl.ds((fbase + g) * NP + s * NPT, NPT)], zb)
        pltpu.sync_copy(zb, tabs.at[pl.ds(g * NP + s * NPT, NPT)])
        pltpu.sync_copy(zb, accs.at[pl.ds(g * NP + s * NPT, NPT)])

      plsc.subcore_barrier()

      @pl.loop(0, NCKT)
      def _(i):
        b = s * NCKT + i
        pltpu.sync_copy(row_h.at[b], srcb)
        pltpu.sync_copy(col_h.at[b], dstb)
        pltpu.sync_copy(ew_h.at[b], ewb)
        for g in range(G):
          cps = [
              pltpu.async_copy(tabs.at[pl.ds(g * NP, NP)].at[srcb.at[j]],
                               valb.at[j], sem)
              for j in range(8)
          ]
          for cp in cps:
            cp.wait()
          for q in range(64):
            sl = (q // 8, pl.ds((q % 8) * 16, 16))
            valb.at[*sl][...] = valb.at[*sl][...] * ewb.at[*sl][...]
          for j in range(8):
            pltpu.sync_copy(valb.at[j],
                            accs.at[pl.ds(g * NP, NP)].at[dstb.at[j]],
                            add=True)

      plsc.subcore_barrier()

      for g in range(G):
        pltpu.sync_copy(accs.at[pl.ds(g * NP + s * NPT, NPT)], zb)
        pltpu.sync_copy(zb, out_h.at[pl.ds((fbase + g) * NP + s * NPT, NPT)])

      plsc.subcore_barrier()

  return k(tabT, row3d, col3d, ew3d)


_HI = jax.lax.Precision.HIGHEST


def _tc_prep_body(degp_ref, x8_ref, dinv_ref, xs8_ref):
  deg = degp_ref[0, :] + 1.0
  dv = jax.lax.rsqrt(deg)
  dinv_ref[...] = dv[:, None]
  xs8_ref[...] = dv[:, None] * x8_ref[...]


def _tc_prep(degp, x8):
  return pl.pallas_call(
      _tc_prep_body,
      grid=(NP // RB,),
      in_specs=[
          pl.BlockSpec((2, RB), lambda i: (0, i)),
          pl.BlockSpec((RB, 8), lambda i: (i, 0)),
      ],
      out_specs=[
          pl.BlockSpec((RB, 1), lambda i: (i, 0)),
          pl.BlockSpec((RB, 8), lambda i: (i, 0)),
      ],
      out_shape=[
          jax.ShapeDtypeStruct((NP, 1), jnp.float32),
          jax.ShapeDtypeStruct((NP, 8), jnp.float32),
      ],
  )(degp, x8)


def _tc_mlp_body(agg1_ref, dinv_ref, w18_ref, b1_ref, w2_ref, we_ref, qs_ref):
  dv = dinv_ref[...]
  u = dv * agg1_ref[...]
  h = jnp.maximum(
      jax.lax.dot_general(u, w18_ref[...], (((1,), (0,)), ((), ())),
                          precision=_HI) + b1_ref[...], 0.0)
  w2e = jax.lax.dot_general(w2_ref[...], we_ref[...], (((1,), (0,)), ((), ())),
                            precision=_HI)
  q = jax.lax.dot_general(h, w2e, (((1,), (0,)), ((), ())), precision=_HI)
  qs_ref[...] = dv * q


def _tc_mlp(agg1, dinv, w18, b1, w2, we):
  return pl.pallas_call(
      _tc_mlp_body,
      grid=(NP // RB,),
      in_specs=[
          pl.BlockSpec((RB, 8), lambda i: (i, 0)),
          pl.BlockSpec((RB, 1), lambda i: (i, 0)),
          pl.BlockSpec((8, 320), lambda i: (0, 0)),
          pl.BlockSpec((1, 320), lambda i: (0, 0)),
          pl.BlockSpec((320, 160), lambda i: (0, 0)),
          pl.BlockSpec((160, 80), lambda i: (0, 0)),
      ],
      out_specs=pl.BlockSpec((RB, 80), lambda i: (i, 0)),
      out_shape=jax.ShapeDtypeStruct((NP, 80), jnp.float32),
  )(agg1, dinv, w18, b1, w2, we)


def _tc_out_body(agg2_ref, dinv_ref, b2_ref, we_ref, be_ref, out_ref):
  bconst = jax.lax.dot_general(b2_ref[...], we_ref[...],
                               (((1,), (0,)), ((), ())),
                               precision=_HI) + be_ref[...]
  out_ref[...] = dinv_ref[...] * agg2_ref[...] + bconst


def _tc_out(agg2, dinv, b2, we, be):
  return pl.pallas_call(
      _tc_out_body,
      grid=(NP // RB,),
      in_specs=[
          pl.BlockSpec((RB, 80), lambda i: (i, 0)),
          pl.BlockSpec((RB, 1), lambda i: (i, 0)),
          pl.BlockSpec((1, 160), lambda i: (0, 0)),
          pl.BlockSpec((160, 80), lambda i: (0, 0)),
          pl.BlockSpec((1, 80), lambda i: (0, 0)),
      ],
      out_specs=pl.BlockSpec((RB, 80), lambda i: (i, 0)),
      out_shape=jax.ShapeDtypeStruct((NP, 80), jnp.float32),
  )(agg2, dinv, b2, we, be)


def kernel(x, edge_index, edge_weight, W1, b1, W2, b2, We, be):
  row = edge_index[0].astype(jnp.int32)
  col = edge_index[1].astype(jnp.int32)
  rowp = jnp.pad(row, (0, EP - E)).reshape(NCK, 8, 128)
  colp = jnp.pad(col, (0, EP - E)).reshape(NCK, 8, 128)
  ewp = jnp.pad(edge_weight, (0, EP - E)).reshape(NCK, 8, 128)
  x8 = jnp.pad(x, ((0, NP - N), (0, 3)))
  w18 = jnp.pad(W1, ((0, 3), (0, 0)))

  degp = _sc_deg(colp, ewp).reshape(2, NP)
  dinv, xs8 = _tc_prep(degp, x8)
  agg1 = _sc_agg(4, 4, xs8.T.reshape(-1), rowp, colp, ewp)
  qs = _tc_mlp(agg1.reshape(8, NP).T, dinv, w18, b1.reshape(1, 320), W2, We)
  agg2 = _sc_agg(40, 8, qs.T.reshape(-1), rowp, colp, ewp)
  outp = _tc_out(agg2.reshape(80, NP).T, dinv, b2.reshape(1, 160),
                 We, be.reshape(1, 80))
  return outp[:N]


# speedup vs baseline: 9.6809x; 1.3948x over previous
import dataclasses
import functools

import jax
import jax.numpy as jnp
from jax import lax
from jax.experimental import pallas as pl
from jax.experimental.pallas import tpu as pltpu
from jax.experimental.pallas import tpu_sc as plsc

N = 50000
NP = 50176
NPT = NP // 16
E = 800000
EP = 802816
NCK = EP // 1024
NCKT = NCK // 16
RB = 1024

_mesh = plsc.VectorSubcoreMesh(core_axis_name="c", subcore_axis_name="s")

_sc_params = pltpu.CompilerParams()
if "needs_layout_passes" in pltpu.CompilerParams.__dataclass_fields__:
  _sc_params = dataclasses.replace(_sc_params, needs_layout_passes=False)


def _sc_deg(col3d, ew3d):
  @functools.partial(
      pl.kernel,
      out_type=jax.ShapeDtypeStruct((2 * NP,), jnp.float32),
      mesh=_mesh,
      scratch_types=[
          pltpu.VMEM((8, 128), jnp.int32),
          pltpu.VMEM((8, 128), jnp.float32),
          pltpu.VMEM((NPT,), jnp.float32),
          pltpu.VMEM_SHARED((NP,), jnp.float32),
      ],
      compiler_params=_sc_params,
  )
  def k(col_h, ew_h, out_h, colb, ewb, zb, acc):
    c = lax.axis_index("c")
    s = lax.axis_index("s")
    zv = jnp.zeros((16,), jnp.float32)

    @pl.loop(0, NPT // 16)
    def _(i):
      zb.at[pl.ds(i * 16, 16)][...] = zv

    pltpu.sync_copy(zb, acc.at[pl.ds(s * NPT, NPT)])
    plsc.subcore_barrier()

    @pl.loop(0, NCKT)
    def _(i):
      b = s * NCKT + i
      pltpu.sync_copy(col_h.at[b], colb)
      pltpu.sync_copy(ew_h.at[b], ewb)
      for j in range(8):
        pltpu.sync_copy(ewb.at[j], acc.at[colb.at[j]], add=True)

    plsc.subcore_barrier()
    pltpu.sync_copy(acc.at[pl.ds(s * NPT, NPT)], zb)
    pltpu.sync_copy(zb, out_h.at[pl.ds(c * NP + s * NPT, NPT)])

  return k(col3d, ew3d)


def _sc_agg(FPC, G, tabT, row3d, col3d, ew3d):
  NG = FPC // G

  @functools.partial(
      pl.kernel,
      out_type=jax.ShapeDtypeStruct((2 * FPC * NP,), jnp.float32),
      mesh=_mesh,
      scratch_types=[
          pltpu.VMEM((8, 128), jnp.int32),
          pltpu.VMEM((8, 128), jnp.int32),
          pltpu.VMEM((8, 128), jnp.float32),
          pltpu.VMEM((16, 128), jnp.float32),
          pltpu.VMEM((NPT,), jnp.float32),
          pltpu.VMEM_SHARED((G * NP,), jnp.float32),
          pltpu.VMEM_SHARED((G * NP,), jnp.float32),
          pltpu.SemaphoreType.DMA,
          pltpu.SemaphoreType.DMA,
      ],
      compiler_params=_sc_params,
  )
  def k(tab_h, row_h, col_h, ew_h, out_h, srcb, dstb, ewb, valb, zb, tabs,
        accs, sem, sem2):
    c = lax.axis_index("c")
    s = lax.axis_index("s")

    @pl.loop(0, NG)
    def _(ng):
      fbase = c * FPC + ng * G
      for g in range(G):
        pltpu.sync_copy(tab_h.at[pl.ds((fbase + g) * NP + s * NPT, NPT)], zb)
        pltpu.sync_copy(zb, tabs.at[pl.ds(g * NP + s * NPT, NPT)])
        pltpu.sync_copy(zb, accs.at[pl.ds(g * NP + s * NPT, NPT)])

      plsc.subcore_barrier()

      @pl.loop(0, NCKT)
      def _(i):
        b = s * NCKT + i
        pltpu.sync_copy(row_h.at[b], srcb)
        pltpu.sync_copy(col_h.at[b], dstb)
        pltpu.sync_copy(ew_h.at[b], ewb)
        pend = [None, None]
        for g in range(G):
          hb = (g % 2) * 8
          if pend[g % 2] is not None:
            for cp in pend[g % 2]:
              cp.wait()
          cps = [
              pltpu.async_copy(tabs.at[pl.ds(g * NP, NP)].at[srcb.at[j]],
                               valb.at[hb + j], sem)
              for j in range(8)
          ]
          for cp in cps:
            cp.wait()
          for q in range(64):
            slv = (hb + q // 8, pl.ds((q % 8) * 16, 16))
            slw = (q // 8, pl.ds((q % 8) * 16, 16))
            valb.at[*slv][...] = valb.at[*slv][...] * ewb.at[*slw][...]
          pend[g % 2] = [
              pltpu.async_copy(valb.at[hb + j],
                               accs.at[pl.ds(g * NP, NP)].at[dstb.at[j]],
                               sem2, add=True)
              for j in range(8)
          ]
        for p2 in pend:
          if p2 is not None:
            for cp in p2:
              cp.wait()

      plsc.subcore_barrier()

      for g in range(G):
        pltpu.sync_copy(accs.at[pl.ds(g * NP + s * NPT, NPT)], zb)
        pltpu.sync_copy(zb, out_h.at[pl.ds((fbase + g) * NP + s * NPT, NPT)])

      plsc.subcore_barrier()

  return k(tabT, row3d, col3d, ew3d)


_HI = jax.lax.Precision.HIGHEST


def _tc_prep_body(degp_ref, x8_ref, dinv_ref, xs8_ref):
  deg = degp_ref[0, :] + 1.0
  dv = jax.lax.rsqrt(deg)
  dinv_ref[...] = dv[:, None]
  xs8_ref[...] = dv[:, None] * x8_ref[...]


def _tc_prep(degp, x8):
  return pl.pallas_call(
      _tc_prep_body,
      grid=(NP // RB,),
      in_specs=[
          pl.BlockSpec((2, RB), lambda i: (0, i)),
          pl.BlockSpec((RB, 8), lambda i: (i, 0)),
      ],
      out_specs=[
          pl.BlockSpec((RB, 1), lambda i: (i, 0)),
          pl.BlockSpec((RB, 8), lambda i: (i, 0)),
      ],
      out_shape=[
          jax.ShapeDtypeStruct((NP, 1), jnp.float32),
          jax.ShapeDtypeStruct((NP, 8), jnp.float32),
      ],
  )(degp, x8)


def _tc_mlp_body(agg1_ref, dinv_ref, w18_ref, b1_ref, w2_ref, we_ref, qs_ref):
  dv = dinv_ref[...]
  u = dv * agg1_ref[...]
  h = jnp.maximum(
      jax.lax.dot_general(u, w18_ref[...], (((1,), (0,)), ((), ())),
                          precision=_HI) + b1_ref[...], 0.0)
  w2e = jax.lax.dot_general(w2_ref[...], we_ref[...], (((1,), (0,)), ((), ())),
                            precision=_HI)
  q = jax.lax.dot_general(h, w2e, (((1,), (0,)), ((), ())), precision=_HI)
  qs_ref[...] = dv * q


def _tc_mlp(agg1, dinv, w18, b1, w2, we):
  return pl.pallas_call(
      _tc_mlp_body,
      grid=(NP // RB,),
      in_specs=[
          pl.BlockSpec((RB, 8), lambda i: (i, 0)),
          pl.BlockSpec((RB, 1), lambda i: (i, 0)),
          pl.BlockSpec((8, 320), lambda i: (0, 0)),
          pl.BlockSpec((1, 320), lambda i: (0, 0)),
          pl.BlockSpec((320, 160), lambda i: (0, 0)),
          pl.BlockSpec((160, 80), lambda i: (0, 0)),
      ],
      out_specs=pl.BlockSpec((RB, 80), lambda i: (i, 0)),
      out_shape=jax.ShapeDtypeStruct((NP, 80), jnp.float32),
  )(agg1, dinv, w18, b1, w2, we)


def _tc_out_body(agg2_ref, dinv_ref, b2_ref, we_ref, be_ref, out_ref):
  bconst = jax.lax.dot_general(b2_ref[...], we_ref[...],
                               (((1,), (0,)), ((), ())),
                               precision=_HI) + be_ref[...]
  out_ref[...] = dinv_ref[...] * agg2_ref[...] + bconst


def _tc_out(agg2, dinv, b2, we, be):
  return pl.pallas_call(
      _tc_out_body,
      grid=(NP // RB,),
      in_specs=[
          pl.BlockSpec((RB, 80), lambda i: (i, 0)),
          pl.BlockSpec((RB, 1), lambda i: (i, 0)),
          pl.BlockSpec((1, 160), lambda i: (0, 0)),
          pl.BlockSpec((160, 80), lambda i: (0, 0)),
          pl.BlockSpec((1, 80), lambda i: (0, 0)),
      ],
      out_specs=pl.BlockSpec((RB, 80), lambda i: (i, 0)),
      out_shape=jax.ShapeDtypeStruct((NP, 80), jnp.float32),
  )(agg2, dinv, b2, we, be)


def kernel(x, edge_index, edge_weight, W1, b1, W2, b2, We, be):
  row = edge_index[0].astype(jnp.int32)
  col = edge_index[1].astype(jnp.int32)
  rowp = jnp.pad(row, (0, EP - E)).reshape(NCK, 8, 128)
  colp = jnp.pad(col, (0, EP - E)).reshape(NCK, 8, 128)
  ewp = jnp.pad(edge_weight, (0, EP - E)).reshape(NCK, 8, 128)
  x8 = jnp.pad(x, ((0, NP - N), (0, 3)))
  w18 = jnp.pad(W1, ((0, 3), (0, 0)))

  degp = _sc_deg(colp, ewp).reshape(2, NP)
  dinv, xs8 = _tc_prep(degp, x8)
  agg1 = _sc_agg(4, 4, xs8.T.reshape(-1), rowp, colp, ewp)
  qs = _tc_mlp(agg1.reshape(8, NP).T, dinv, w18, b1.reshape(1, 320), W2, We)
  agg2 = _sc_agg(40, 8, qs.T.reshape(-1), rowp, colp, ewp)
  outp = _tc_out(agg2.reshape(80, NP).T, dinv, b2.reshape(1, 160),
                 We, be.reshape(1, 80))
  return outp[:N]


# speedup vs baseline: 12.1901x; 1.2592x over previous
import dataclasses
import functools

import jax
import jax.numpy as jnp
from jax import lax
from jax.experimental import pallas as pl
from jax.experimental.pallas import tpu as pltpu
from jax.experimental.pallas import tpu_sc as plsc

N = 50000
NP = 50176
NPT = NP // 16
E = 800000
EP = 802816
NCK = EP // 1024
NCKT = NCK // 16
RB = 1024

_mesh = plsc.VectorSubcoreMesh(core_axis_name="c", subcore_axis_name="s")

_sc_params = pltpu.CompilerParams()
if "needs_layout_passes" in pltpu.CompilerParams.__dataclass_fields__:
  _sc_params = dataclasses.replace(_sc_params, needs_layout_passes=False)


def _sc_deg(col3d, ew3d):
  @functools.partial(
      pl.kernel,
      out_type=jax.ShapeDtypeStruct((2 * NP,), jnp.float32),
      mesh=_mesh,
      scratch_types=[
          pltpu.VMEM((8, 128), jnp.int32),
          pltpu.VMEM((8, 128), jnp.float32),
          pltpu.VMEM((NPT,), jnp.float32),
          pltpu.VMEM_SHARED((NP,), jnp.float32),
      ],
      compiler_params=_sc_params,
  )
  def k(col_h, ew_h, out_h, colb, ewb, zb, acc):
    c = lax.axis_index("c")
    s = lax.axis_index("s")
    zv = jnp.zeros((16,), jnp.float32)

    @pl.loop(0, NPT // 16)
    def _(i):
      zb.at[pl.ds(i * 16, 16)][...] = zv

    pltpu.sync_copy(zb, acc.at[pl.ds(s * NPT, NPT)])
    plsc.subcore_barrier()

    @pl.loop(0, NCKT)
    def _(i):
      b = s * NCKT + i
      pltpu.sync_copy(col_h.at[b], colb)
      pltpu.sync_copy(ew_h.at[b], ewb)
      for j in range(8):
        pltpu.sync_copy(ewb.at[j], acc.at[colb.at[j]], add=True)

    plsc.subcore_barrier()
    pltpu.sync_copy(acc.at[pl.ds(s * NPT, NPT)], zb)
    pltpu.sync_copy(zb, out_h.at[pl.ds(c * NP + s * NPT, NPT)])

  return k(col3d, ew3d)


def _sc_agg(FPC, G, tabT, row3d, col3d, ew3d):
  NG = FPC // G

  @functools.partial(
      pl.kernel,
      out_type=jax.ShapeDtypeStruct((2 * FPC * NP,), jnp.float32),
      mesh=_mesh,
      scratch_types=[
          pltpu.VMEM((8, 128), jnp.int32),
          pltpu.VMEM((8, 128), jnp.int32),
          pltpu.VMEM((8, 128), jnp.float32),
          pltpu.VMEM((24, 128), jnp.float32),
          pltpu.VMEM((NPT,), jnp.float32),
          pltpu.VMEM_SHARED((G * NP,), jnp.float32),
          pltpu.VMEM_SHARED((G * NP,), jnp.float32),
          pltpu.SemaphoreType.DMA,
          pltpu.SemaphoreType.DMA,
      ],
      compiler_params=_sc_params,
  )
  def k(tab_h, row_h, col_h, ew_h, out_h, srcb, dstb, ewb, valb, zb, tabs,
        accs, sem, sem2):
    c = lax.axis_index("c")
    s = lax.axis_index("s")

    @pl.loop(0, NG)
    def _(ng):
      fbase = c * FPC + ng * G
      for g in range(G):
        pltpu.sync_copy(tab_h.at[pl.ds((fbase + g) * NP + s * NPT, NPT)], zb)
        pltpu.sync_copy(zb, tabs.at[pl.ds(g * NP + s * NPT, NPT)])
        pltpu.sync_copy(zb, accs.at[pl.ds(g * NP + s * NPT, NPT)])

      plsc.subcore_barrier()

      @pl.loop(0, NCKT)
      def _(i):
        b = s * NCKT + i
        cpe = [pltpu.async_copy(row_h.at[b], srcb, sem),
               pltpu.async_copy(col_h.at[b], dstb, sem),
               pltpu.async_copy(ew_h.at[b], ewb, sem)]
        for cp in cpe:
          cp.wait()

        def fire_gather(g):
          hb = (g % 3) * 8
          return [
              pltpu.async_copy(tabs.at[pl.ds(g * NP, NP)].at[srcb.at[j]],
                               valb.at[hb + j], sem)
              for j in range(8)
          ]

        pend = [None, None, None]
        gath = [None, None, None]
        gath[0] = fire_gather(0)
        for g in range(G):
          h = g % 3
          hb = h * 8
          for cp in gath[h]:
            cp.wait()
          if g + 1 < G:
            hn = (g + 1) % 3
            if pend[hn] is not None:
              for cp in pend[hn]:
                cp.wait()
              pend[hn] = None
            gath[hn] = fire_gather(g + 1)
          for q in range(64):
            slv = (hb + q // 8, pl.ds((q % 8) * 16, 16))
            slw = (q // 8, pl.ds((q % 8) * 16, 16))
            valb.at[*slv][...] = valb.at[*slv][...] * ewb.at[*slw][...]
          pend[h] = [
              pltpu.async_copy(valb.at[hb + j],
                               accs.at[pl.ds(g * NP, NP)].at[dstb.at[j]],
                               sem2, add=True)
              for j in range(8)
          ]
        for p2 in pend:
          if p2 is not None:
            for cp in p2:
              cp.wait()

      plsc.subcore_barrier()

      for g in range(G):
        pltpu.sync_copy(accs.at[pl.ds(g * NP + s * NPT, NPT)], zb)
        pltpu.sync_copy(zb, out_h.at[pl.ds((fbase + g) * NP + s * NPT, NPT)])

      plsc.subcore_barrier()

  return k(tabT, row3d, col3d, ew3d)


_HI = jax.lax.Precision.HIGHEST


def _tc_prep_body(degp_ref, x8_ref, dinv_ref, xs8_ref):
  deg = degp_ref[0, :] + 1.0
  dv = jax.lax.rsqrt(deg)
  dinv_ref[...] = dv[:, None]
  xs8_ref[...] = dv[:, None] * x8_ref[...]


def _tc_prep(degp, x8):
  return pl.pallas_call(
      _tc_prep_body,
      grid=(NP // RB,),
      in_specs=[
          pl.BlockSpec((2, RB), lambda i: (0, i)),
          pl.BlockSpec((RB, 8), lambda i: (i, 0)),
      ],
      out_specs=[
          pl.BlockSpec((RB, 1), lambda i: (i, 0)),
          pl.BlockSpec((RB, 8), lambda i: (i, 0)),
      ],
      out_shape=[
          jax.ShapeDtypeStruct((NP, 1), jnp.float32),
          jax.ShapeDtypeStruct((NP, 8), jnp.float32),
      ],
  )(degp, x8)


def _tc_mlp_body(agg1_ref, dinv_ref, w18_ref, b1_ref, w2_ref, we_ref, qs_ref):
  dv = dinv_ref[...]
  u = dv * agg1_ref[...]
  h = jnp.maximum(
      jax.lax.dot_general(u, w18_ref[...], (((1,), (0,)), ((), ())),
                          precision=_HI) + b1_ref[...], 0.0)
  w2e = jax.lax.dot_general(w2_ref[...], we_ref[...], (((1,), (0,)), ((), ())),
                            precision=_HI)
  q = jax.lax.dot_general(h, w2e, (((1,), (0,)), ((), ())), precision=_HI)
  qs_ref[...] = dv * q


def _tc_mlp(agg1, dinv, w18, b1, w2, we):
  return pl.pallas_call(
      _tc_mlp_body,
      grid=(NP // RB,),
      in_specs=[
          pl.BlockSpec((RB, 8), lambda i: (i, 0)),
          pl.BlockSpec((RB, 1), lambda i: (i, 0)),
          pl.BlockSpec((8, 320), lambda i: (0, 0)),
          pl.BlockSpec((1, 320), lambda i: (0, 0)),
          pl.BlockSpec((320, 160), lambda i: (0, 0)),
          pl.BlockSpec((160, 80), lambda i: (0, 0)),
      ],
      out_specs=pl.BlockSpec((RB, 80), lambda i: (i, 0)),
      out_shape=jax.ShapeDtypeStruct((NP, 80), jnp.float32),
  )(agg1, dinv, w18, b1, w2, we)


def _tc_out_body(agg2_ref, dinv_ref, b2_ref, we_ref, be_ref, out_ref):
  bconst = jax.lax.dot_general(b2_ref[...], we_ref[...],
                               (((1,), (0,)), ((), ())),
                               precision=_HI) + be_ref[...]
  out_ref[...] = dinv_ref[...] * agg2_ref[...] + bconst


def _tc_out(agg2, dinv, b2, we, be):
  return pl.pallas_call(
      _tc_out_body,
      grid=(NP // RB,),
      in_specs=[
          pl.BlockSpec((RB, 80), lambda i: (i, 0)),
          pl.BlockSpec((RB, 1), lambda i: (i, 0)),
          pl.BlockSpec((1, 160), lambda i: (0, 0)),
          pl.BlockSpec((160, 80), lambda i: (0, 0)),
          pl.BlockSpec((1, 80), lambda i: (0, 0)),
      ],
      out_specs=pl.BlockSpec((RB, 80), lambda i: (i, 0)),
      out_shape=jax.ShapeDtypeStruct((NP, 80), jnp.float32),
  )(agg2, dinv, b2, we, be)


def kernel(x, edge_index, edge_weight, W1, b1, W2, b2, We, be):
  row = edge_index[0].astype(jnp.int32)
  col = edge_index[1].astype(jnp.int32)
  rowp = jnp.pad(row, (0, EP - E)).reshape(NCK, 8, 128)
  colp = jnp.pad(col, (0, EP - E)).reshape(NCK, 8, 128)
  ewp = jnp.pad(edge_weight, (0, EP - E)).reshape(NCK, 8, 128)
  x8 = jnp.pad(x, ((0, NP - N), (0, 3)))
  w18 = jnp.pad(W1, ((0, 3), (0, 0)))

  degp = _sc_deg(colp, ewp).reshape(2, NP)
  dinv, xs8 = _tc_prep(degp, x8)
  agg1 = _sc_agg(4, 4, xs8.T.reshape(-1), rowp, colp, ewp)
  qs = _tc_mlp(agg1.reshape(8, NP).T, dinv, w18, b1.reshape(1, 320), W2, We)
  agg2 = _sc_agg(40, 8, qs.T.reshape(-1), rowp, colp, ewp)
  outp = _tc_out(agg2.reshape(80, NP).T, dinv, b2.reshape(1, 160),
                 We, be.reshape(1, 80))
  return outp[:N]
